# baseline (speedup 1.0000x reference)
import functools

import jax
import jax.numpy as jnp
from jax import lax
from jax.experimental import pallas as pl
from jax.experimental.pallas import tpu as pltpu
from jax.experimental.pallas import tpu_sc as plsc

N = 10000
NP = 10240
E = 160000
EP = 163840
DC = 64
H = 512
C = 128
G = 64
EPS = 1e-5

NTILES = 32
EPT = EP // NTILES
CH = 128
NCH = EPT // CH
RPT = NP // 16
RB = 1280
NGRID = NP // RB


NB = 4
NG = NCH // NB
MASK_HI = -65536


def _sc_agg(tables, srcp2d, dstp2d, zeros2d, zeros1d, with_deg):
  F = len(tables)
  mesh = plsc.VectorSubcoreMesh(core_axis_name="c", subcore_axis_name="s")
  out_type = [jax.ShapeDtypeStruct((2 * NP, DC), jnp.float32) for _ in range(F)]
  if with_deg:
    out_type.append(jax.ShapeDtypeStruct((2 * NP,), jnp.float32))

  scratch_types = (
      [pltpu.VMEM((NCH, CH), jnp.int32),
       pltpu.VMEM((NCH, CH), jnp.int32),
       pltpu.VMEM((CH,), jnp.float32)]
      + [pltpu.VMEM((CH, DC), jnp.bfloat16) for _ in range(NB)]
      + [pltpu.VMEM((CH, DC), jnp.float32) for _ in range(NB)]
      + [pltpu.VMEM_SHARED((NP, DC), jnp.float32),
         pltpu.VMEM_SHARED((NP,), jnp.float32)]
      + [pltpu.SemaphoreType.DMA for _ in range(NB)]
      + [pltpu.SemaphoreType.DMA,
         pltpu.SemaphoreType.DMA]
  )

  def body(*refs):
    tabs = refs[:F]
    srcr, dstr, z2, z1 = refs[F:F + 4]
    nin = F + 4
    outs = refs[nin:nin + F]
    degout = refs[nin + F] if with_deg else None
    sc = refs[nin + F + (1 if with_deg else 0):]
    sidx_all, didx_all, ones_v = sc[0:3]
    rows_bf = sc[3:3 + NB]
    rows = sc[3 + NB:3 + 2 * NB]
    agg_sh, deg_sh = sc[3 + 2 * NB:5 + 2 * NB]
    gsems = sc[5 + 2 * NB:5 + 3 * NB]
    ssem, dsem = sc[5 + 3 * NB:7 + 3 * NB]

    cid = lax.axis_index("c")
    sid = lax.axis_index("s")
    wid = sid * 2 + cid
    r0 = sid * RPT

    pltpu.sync_copy(srcr.at[pl.ds(wid * NCH, NCH)], sidx_all)
    pltpu.sync_copy(dstr.at[pl.ds(wid * NCH, NCH)], didx_all)
    if with_deg:
      for k in range(CH // 16):
        ones_v[pl.ds(k * 16, 16)] = jnp.full((16,), 1.0, jnp.float32)

    for f in range(F):
      pltpu.sync_copy(z2.at[pl.ds(r0, RPT)], agg_sh.at[pl.ds(r0, RPT)])
      if with_deg and f == 0:
        pltpu.sync_copy(z1.at[pl.ds(r0, RPT)], deg_sh.at[pl.ds(r0, RPT)])
      plsc.subcore_barrier()

      def group(g, carry):
        gd = []
        for b in range(NB):
          c = g * NB + b
          gd.append(pltpu.async_copy(tabs[f].at[sidx_all.at[c]], rows_bf[b],
                                     gsems[b]))
        sd = []
        for b in range(NB):
          c = g * NB + b
          gd[b].wait()

          def conv(r4, cc):
            for r0 in range(4):
              for k in range(DC // 32):
                v = rows_bf[b][r4 * 4 + r0, pl.ds(32 * k, 32)]
                vi = plsc.bitcast(v, jnp.int32)
                lo = plsc.bitcast(vi << 16, jnp.float32)
                hi = plsc.bitcast(vi & MASK_HI, jnp.float32)
                rows[b][r4 * 4 + r0, pl.ds(32 * k, 16)] = lo
                rows[b][r4 * 4 + r0, pl.ds(32 * k + 16, 16)] = hi
            return cc
          lax.fori_loop(0, CH // 4, conv, 0)

          sd.append(pltpu.async_copy(rows[b], agg_sh.at[didx_all.at[c]],
                                     ssem, add=True))
          if with_deg and f == 0:
            sd.append(pltpu.async_copy(ones_v, deg_sh.at[didx_all.at[c]],
                                       dsem, add=True))
        for d in sd:
          d.wait()
        return carry

      lax.fori_loop(0, NG, group, 0)
      plsc.subcore_barrier()

      pltpu.sync_copy(agg_sh.at[pl.ds(r0, RPT)],
                      outs[f].at[pl.ds(cid * NP + r0, RPT)])
      if with_deg and f == 0:
        pltpu.sync_copy(deg_sh.at[pl.ds(r0, RPT)],
                        degout.at[pl.ds(cid * NP + r0, RPT)])

  fn = pl.kernel(body, out_type=out_type, mesh=mesh,
                 scratch_types=scratch_types,
                 compiler_params=pltpu.CompilerParams(
                     use_tc_tiling_on_sc=False, needs_layout_passes=False))
  res = fn(*tables, srcp2d, dstp2d, zeros2d, zeros1d)
  if with_deg:
    parts = [r.reshape(2, NP, DC) for r in res[:F]]
    return parts, res[F].reshape(2, NP)
  return [r.reshape(2, NP, DC) for r in res]


def _linear_body(F, *refs):
  parts = refs[:F]
  deg = refs[F]
  h_ref = refs[F + 1]
  wl, wr, b = refs[F + 2:F + 5]
  z_ref, st_ref = refs[F + 5:]

  i = pl.program_id(0)
  d = deg[0, :] + deg[1, :]
  inv = 1.0 / jnp.maximum(d, 1.0)

  dn = (((1,), (1,)), ((), ()))
  pre = jnp.concatenate([parts[f][0] + parts[f][1] for f in range(F)],
                        axis=1) * inv[:, None]
  hcat = h_ref[...]
  acc = (jnp.broadcast_to(b[...][None, :], (RB, H)).astype(jnp.float32)
         + lax.dot_general(pre, wl[...], dn,
                           preferred_element_type=jnp.float32)
         + lax.dot_general(hcat, wr[...], dn,
                           preferred_element_type=jnp.float32))

  rows = i * RB + lax.broadcasted_iota(jnp.int32, (RB, 1), 0)
  z = jnp.where(rows < N, acc, 0.0)
  z_ref[...] = z

  s1 = jnp.sum(z, axis=0)
  s2 = jnp.sum(z * z, axis=0)

  @pl.when(i == 0)
  def _():
    st_ref[...] = jnp.zeros((2, H), jnp.float32)

  st_ref[0, :] += s1
  st_ref[1, :] += s2


def _tc_linear(parts, deg, h, Wl, Wr, b):
  F = len(parts)
  in_specs = (
      [pl.BlockSpec((2, RB, DC), lambda i: (0, i, 0)) for _ in range(F)]
      + [pl.BlockSpec((2, RB), lambda i: (0, i))]
      + [pl.BlockSpec((RB, F * DC), lambda i: (i, 0))]
      + [pl.BlockSpec((H, F * DC), lambda i: (0, 0)),
         pl.BlockSpec((H, F * DC), lambda i: (0, 0)),
         pl.BlockSpec((H,), lambda i: (0,))]
  )
  out_specs = [pl.BlockSpec((RB, H), lambda i: (i, 0)),
               pl.BlockSpec((2, H), lambda i: (0, 0))]
  return pl.pallas_call(
      functools.partial(_linear_body, F),
      grid=(NGRID,),
      in_specs=in_specs,
      out_specs=out_specs,
      out_shape=[jax.ShapeDtypeStruct((NP, H), jnp.float32),
                 jax.ShapeDtypeStruct((2, H), jnp.float32)],
  )(*parts, deg, h, Wl, Wr, b)


def _bn_body(z_ref, st_ref, g_ref, be_ref, h_ref, *out_refs):
  mu = st_ref[0, :] / N
  var = st_ref[1, :] / N - mu * mu
  rstd = lax.rsqrt(var + EPS)
  h = jnp.maximum((z_ref[...] - mu[None, :]) * (rstd * g_ref[...])[None, :]
                  + be_ref[...][None, :], 0.0)
  h_ref[...] = h
  hp = h.astype(jnp.bfloat16)
  for f, o in enumerate(out_refs):
    o[...] = hp[:, f * DC:(f + 1) * DC]


def _tc_bn_relu(z, st, g, be):
  F = H // DC
  return pl.pallas_call(
      _bn_body,
      grid=(NGRID,),
      in_specs=[pl.BlockSpec((RB, H), lambda i: (i, 0)),
                pl.BlockSpec((2, H), lambda i: (0, 0)),
                pl.BlockSpec((H,), lambda i: (0,)),
                pl.BlockSpec((H,), lambda i: (0,))],
      out_specs=([pl.BlockSpec((RB, H), lambda i: (i, 0))]
                 + [pl.BlockSpec((RB, DC), lambda i: (i, 0))
                    for _ in range(F)]),
      out_shape=([jax.ShapeDtypeStruct((NP, H), jnp.float32)]
                 + [jax.ShapeDtypeStruct((NP, DC), jnp.bfloat16)
                    for _ in range(F)]),
  )(z, st, g, be)


def _pool_body(z_ref, st_ref, g_ref, be_ref, b_ref, w1_ref, b1_ref, w2_ref,
               b2_ref, out_ref, pool_ref, cnt_ref):
  i = pl.program_id(0)
  mu = st_ref[0, :] / N
  var = st_ref[1, :] / N - mu * mu
  rstd = lax.rsqrt(var + EPS)
  h = jnp.maximum((z_ref[...] - mu[None, :]) * (rstd * g_ref[...])[None, :]
                  + be_ref[...][None, :], 0.0)

  bvals = b_ref[0, 0, :]
  gids = lax.broadcasted_iota(jnp.int32, (G, RB), 0).astype(jnp.float32)
  oh = jnp.where(gids == bvals[None, :], 1.0, 0.0)

  @pl.when(i == 0)
  def _():
    pool_ref[...] = jnp.zeros((G, H), jnp.float32)
    cnt_ref[...] = jnp.zeros((G, 128), jnp.float32)

  dn = (((1,), (0,)), ((), ()))
  pool_ref[...] += lax.dot_general(oh, h, dn,
                                   preferred_element_type=jnp.float32)
  cnt_ref[...] += jnp.broadcast_to(jnp.sum(oh, axis=1)[:, None], (G, 128))

  @pl.when(i == NGRID - 1)
  def _():
    cnt = cnt_ref[:, 0:1]
    pooled = pool_ref[...] / jnp.maximum(cnt, 1.0)
    dnt = (((1,), (1,)), ((), ()))
    t = jnp.maximum(
        lax.dot_general(pooled, w1_ref[...], dnt,
                        preferred_element_type=jnp.float32)
        + b1_ref[...][None, :], 0.0)
    out_ref[...] = (lax.dot_general(t, w2_ref[...], dnt,
                                    preferred_element_type=jnp.float32)
                    + b2_ref[...][None, :])


def _tc_pool_head(z, st, g, be, batchf, Wfc1, bfc1, Wfc2, bfc2):
  return pl.pallas_call(
      _pool_body,
      grid=(NGRID,),
      in_specs=[pl.BlockSpec((RB, H), lambda i: (i, 0)),
                pl.BlockSpec((2, H), lambda i: (0, 0)),
                pl.BlockSpec((H,), lambda i: (0,)),
                pl.BlockSpec((H,), lambda i: (0,)),
                pl.BlockSpec((1, 1, RB), lambda i: (i, 0, 0)),
                pl.BlockSpec((H, H), lambda i: (0, 0)),
                pl.BlockSpec((H,), lambda i: (0,)),
                pl.BlockSpec((C, H), lambda i: (0, 0)),
                pl.BlockSpec((C,), lambda i: (0,))],
      out_specs=pl.BlockSpec((G, C), lambda i: (0, 0)),
      out_shape=jax.ShapeDtypeStruct((G, C), jnp.float32),
      scratch_shapes=[pltpu.VMEM((G, H), jnp.float32),
                      pltpu.VMEM((G, 128), jnp.float32)],
  )(z, st, g, be, batchf, Wfc1, bfc1, Wfc2, bfc2)


def kernel(x, edge_index, batch, Wl1, Wr1, b1, g1, be1, Wl2, Wr2, b2, g2, be2,
           Wl3, Wr3, b3, g3, be3, Wfc1, bfc1, Wfc2, bfc2):
  src, dst = lax.sort([edge_index[0], edge_index[1]], num_keys=1)
  srcp = jnp.concatenate([src, jnp.zeros((EP - E,), jnp.int32)]
                         ).reshape(EP // CH, CH)
  dstp = jnp.concatenate([dst, jnp.full((EP - E,), N, jnp.int32)]
                         ).reshape(EP // CH, CH)
  zeros2d = jnp.zeros((NP, DC), jnp.float32)
  zeros1d = jnp.zeros((NP,), jnp.float32)

  D0 = x.shape[1]
  xp = jnp.pad(x, ((0, NP - N), (0, 0)))
  xpm = xp.astype(jnp.bfloat16)
  xb_bf = [xpm[:, f * DC:(f + 1) * DC] for f in range(D0 // DC)]

  def _wl_perm(w):
    d = w.shape[1]
    qg = jnp.arange(32).reshape(16, 2).T.reshape(32)
    idx = (32 * jnp.arange(d // 32)[:, None] + qg[None, :]).reshape(d)
    return w[:, idx]
  Wl1, Wl2, Wl3 = _wl_perm(Wl1), _wl_perm(Wl2), _wl_perm(Wl3)
  batchf = (jnp.pad(batch, (0, NP - N), constant_values=G)
            .astype(jnp.float32).reshape(NGRID, 1, RB))

  parts1, deg = _sc_agg(xb_bf, srcp, dstp, zeros2d, zeros1d, with_deg=True)
  z1, st1 = _tc_linear(parts1, deg, xp, Wl1, Wr1, b1)
  h1f, *h1b = _tc_bn_relu(z1, st1, g1, be1)

  parts2 = _sc_agg(h1b, srcp, dstp, zeros2d, zeros1d, with_deg=False)
  z2, st2 = _tc_linear(parts2, deg, h1f, Wl2, Wr2, b2)
  h2f, *h2b = _tc_bn_relu(z2, st2, g2, be2)

  parts3 = _sc_agg(h2b, srcp, dstp, zeros2d, zeros1d, with_deg=False)
  z3, st3 = _tc_linear(parts3, deg, h2f, Wl3, Wr3, b3)
  return _tc_pool_head(z3, st3, g3, be3, batchf, Wfc1, bfc1, Wfc2, bfc2)


# speedup vs baseline: 1.0837x; 1.0837x over previous
import functools

import jax
import jax.numpy as jnp
from jax import lax
from jax.experimental import pallas as pl
from jax.experimental.pallas import tpu as pltpu
from jax.experimental.pallas import tpu_sc as plsc

N = 10000
NP = 10240
E = 160000
EP = 163840
DC = 128
H = 512
C = 128
G = 64
EPS = 1e-5

NTILES = 32
EPT = EP // NTILES
CH = 128
NCH = EPT // CH
RPT = NP // 16
RB = 1280
NGRID = NP // RB


NB = 2
NG = NCH // NB
MASK_HI = -65536


def _sc_agg(tables, srcp2d, dstp2d, zeros2d, zeros1d, with_deg):
  F = len(tables)
  mesh = plsc.VectorSubcoreMesh(core_axis_name="c", subcore_axis_name="s")
  out_type = [jax.ShapeDtypeStruct((2 * NP, DC), jnp.float32) for _ in range(F)]
  if with_deg:
    out_type.append(jax.ShapeDtypeStruct((2 * NP,), jnp.float32))

  scratch_types = (
      [pltpu.VMEM((NCH, CH), jnp.int32),
       pltpu.VMEM((NCH, CH), jnp.int32),
       pltpu.VMEM((CH,), jnp.float32)]
      + [pltpu.VMEM((CH, DC), jnp.bfloat16) for _ in range(NB)]
      + [pltpu.VMEM((CH, DC), jnp.float32)]
      + [pltpu.VMEM_SHARED((NP, DC), jnp.float32),
         pltpu.VMEM_SHARED((NP,), jnp.float32)]
      + [pltpu.SemaphoreType.DMA for _ in range(NB)]
      + [pltpu.SemaphoreType.DMA,
         pltpu.SemaphoreType.DMA]
  )

  def body(*refs):
    tabs = refs[:F]
    srcr, dstr, z2, z1 = refs[F:F + 4]
    nin = F + 4
    outs = refs[nin:nin + F]
    degout = refs[nin + F] if with_deg else None
    sc = refs[nin + F + (1 if with_deg else 0):]
    sidx_all, didx_all, ones_v = sc[0:3]
    rows_bf = sc[3:3 + NB]
    payload = sc[3 + NB]
    agg_sh, deg_sh = sc[4 + NB:6 + NB]
    gsems = sc[6 + NB:6 + 2 * NB]
    ssem, dsem = sc[6 + 2 * NB:8 + 2 * NB]

    cid = lax.axis_index("c")
    sid = lax.axis_index("s")
    wid = sid * 2 + cid
    r0 = sid * RPT

    pltpu.sync_copy(srcr.at[pl.ds(wid * NCH, NCH)], sidx_all)
    pltpu.sync_copy(dstr.at[pl.ds(wid * NCH, NCH)], didx_all)
    if with_deg:
      for k in range(CH // 16):
        ones_v[pl.ds(k * 16, 16)] = jnp.full((16,), 1.0, jnp.float32)

    for f in range(F):
      pltpu.sync_copy(z2.at[pl.ds(r0, RPT)], agg_sh.at[pl.ds(r0, RPT)])
      if with_deg and f == 0:
        pltpu.sync_copy(z1.at[pl.ds(r0, RPT)], deg_sh.at[pl.ds(r0, RPT)])
      plsc.subcore_barrier()

      def group(g, carry):
        gd = []
        for b in range(NB):
          c = g * NB + b
          gd.append(pltpu.async_copy(tabs[f].at[sidx_all.at[c]], rows_bf[b],
                                     gsems[b]))
        psd = None
        dd = []
        for b in range(NB):
          c = g * NB + b
          gd[b].wait()
          if psd is not None:
            psd.wait()

          def conv(r4, cc):
            for r0 in range(4):
              for k in range(DC // 32):
                v = rows_bf[b][r4 * 4 + r0, pl.ds(32 * k, 32)]
                vi = plsc.bitcast(v, jnp.int32)
                lo = plsc.bitcast(vi << 16, jnp.float32)
                hi = plsc.bitcast(vi & MASK_HI, jnp.float32)
                payload[r4 * 4 + r0, pl.ds(32 * k, 16)] = lo
                payload[r4 * 4 + r0, pl.ds(32 * k + 16, 16)] = hi
            return cc
          lax.fori_loop(0, CH // 4, conv, 0)

          psd = pltpu.async_copy(payload, agg_sh.at[didx_all.at[c]],
                                 ssem, add=True)
          if with_deg and f == 0:
            dd.append(pltpu.async_copy(ones_v, deg_sh.at[didx_all.at[c]],
                                       dsem, add=True))
        psd.wait()
        for d in dd:
          d.wait()
        return carry

      lax.fori_loop(0, NG, group, 0)
      plsc.subcore_barrier()

      pltpu.sync_copy(agg_sh.at[pl.ds(r0, RPT)],
                      outs[f].at[pl.ds(cid * NP + r0, RPT)])
      if with_deg and f == 0:
        pltpu.sync_copy(deg_sh.at[pl.ds(r0, RPT)],
                        degout.at[pl.ds(cid * NP + r0, RPT)])

  fn = pl.kernel(body, out_type=out_type, mesh=mesh,
                 scratch_types=scratch_types,
                 compiler_params=pltpu.CompilerParams(
                     use_tc_tiling_on_sc=False, needs_layout_passes=False))
  res = fn(*tables, srcp2d, dstp2d, zeros2d, zeros1d)
  if with_deg:
    parts = [r.reshape(2, NP, DC) for r in res[:F]]
    return parts, res[F].reshape(2, NP)
  return [r.reshape(2, NP, DC) for r in res]


def _linear_body(F, *refs):
  parts = refs[:F]
  deg = refs[F]
  h_ref = refs[F + 1]
  wl, wr, b = refs[F + 2:F + 5]
  z_ref, st_ref = refs[F + 5:]

  i = pl.program_id(0)
  d = deg[0, :] + deg[1, :]
  inv = 1.0 / jnp.maximum(d, 1.0)

  dn = (((1,), (1,)), ((), ()))
  pre = jnp.concatenate([parts[f][0] + parts[f][1] for f in range(F)],
                        axis=1) * inv[:, None]
  hcat = h_ref[...]
  acc = (jnp.broadcast_to(b[...][None, :], (RB, H)).astype(jnp.float32)
         + lax.dot_general(pre, wl[...], dn,
                           preferred_element_type=jnp.float32)
         + lax.dot_general(hcat, wr[...], dn,
                           preferred_element_type=jnp.float32))

  rows = i * RB + lax.broadcasted_iota(jnp.int32, (RB, 1), 0)
  z = jnp.where(rows < N, acc, 0.0)
  z_ref[...] = z

  s1 = jnp.sum(z, axis=0)
  s2 = jnp.sum(z * z, axis=0)

  @pl.when(i == 0)
  def _():
    st_ref[...] = jnp.zeros((2, H), jnp.float32)

  st_ref[0, :] += s1
  st_ref[1, :] += s2


def _tc_linear(parts, deg, h, Wl, Wr, b):
  F = len(parts)
  in_specs = (
      [pl.BlockSpec((2, RB, DC), lambda i: (0, i, 0)) for _ in range(F)]
      + [pl.BlockSpec((2, RB), lambda i: (0, i))]
      + [pl.BlockSpec((RB, F * DC), lambda i: (i, 0))]
      + [pl.BlockSpec((H, F * DC), lambda i: (0, 0)),
         pl.BlockSpec((H, F * DC), lambda i: (0, 0)),
         pl.BlockSpec((H,), lambda i: (0,))]
  )
  out_specs = [pl.BlockSpec((RB, H), lambda i: (i, 0)),
               pl.BlockSpec((2, H), lambda i: (0, 0))]
  return pl.pallas_call(
      functools.partial(_linear_body, F),
      grid=(NGRID,),
      in_specs=in_specs,
      out_specs=out_specs,
      out_shape=[jax.ShapeDtypeStruct((NP, H), jnp.float32),
                 jax.ShapeDtypeStruct((2, H), jnp.float32)],
  )(*parts, deg, h, Wl, Wr, b)


def _bn_body(z_ref, st_ref, g_ref, be_ref, h_ref, *out_refs):
  mu = st_ref[0, :] / N
  var = st_ref[1, :] / N - mu * mu
  rstd = lax.rsqrt(var + EPS)
  h = jnp.maximum((z_ref[...] - mu[None, :]) * (rstd * g_ref[...])[None, :]
                  + be_ref[...][None, :], 0.0)
  h_ref[...] = h
  hp = h.astype(jnp.bfloat16)
  for f, o in enumerate(out_refs):
    o[...] = hp[:, f * DC:(f + 1) * DC]


def _tc_bn_relu(z, st, g, be):
  F = H // DC
  return pl.pallas_call(
      _bn_body,
      grid=(NGRID,),
      in_specs=[pl.BlockSpec((RB, H), lambda i: (i, 0)),
                pl.BlockSpec((2, H), lambda i: (0, 0)),
                pl.BlockSpec((H,), lambda i: (0,)),
                pl.BlockSpec((H,), lambda i: (0,))],
      out_specs=([pl.BlockSpec((RB, H), lambda i: (i, 0))]
                 + [pl.BlockSpec((RB, DC), lambda i: (i, 0))
                    for _ in range(F)]),
      out_shape=([jax.ShapeDtypeStruct((NP, H), jnp.float32)]
                 + [jax.ShapeDtypeStruct((NP, DC), jnp.bfloat16)
                    for _ in range(F)]),
  )(z, st, g, be)


def _pool_body(z_ref, st_ref, g_ref, be_ref, b_ref, w1_ref, b1_ref, w2_ref,
               b2_ref, out_ref, pool_ref, cnt_ref):
  i = pl.program_id(0)
  mu = st_ref[0, :] / N
  var = st_ref[1, :] / N - mu * mu
  rstd = lax.rsqrt(var + EPS)
  h = jnp.maximum((z_ref[...] - mu[None, :]) * (rstd * g_ref[...])[None, :]
                  + be_ref[...][None, :], 0.0)

  bvals = b_ref[0, 0, :]
  gids = lax.broadcasted_iota(jnp.int32, (G, RB), 0).astype(jnp.float32)
  oh = jnp.where(gids == bvals[None, :], 1.0, 0.0)

  @pl.when(i == 0)
  def _():
    pool_ref[...] = jnp.zeros((G, H), jnp.float32)
    cnt_ref[...] = jnp.zeros((G, 128), jnp.float32)

  dn = (((1,), (0,)), ((), ()))
  pool_ref[...] += lax.dot_general(oh, h, dn,
                                   preferred_element_type=jnp.float32)
  cnt_ref[...] += jnp.broadcast_to(jnp.sum(oh, axis=1)[:, None], (G, 128))

  @pl.when(i == NGRID - 1)
  def _():
    cnt = cnt_ref[:, 0:1]
    pooled = pool_ref[...] / jnp.maximum(cnt, 1.0)
    dnt = (((1,), (1,)), ((), ()))
    t = jnp.maximum(
        lax.dot_general(pooled, w1_ref[...], dnt,
                        preferred_element_type=jnp.float32)
        + b1_ref[...][None, :], 0.0)
    out_ref[...] = (lax.dot_general(t, w2_ref[...], dnt,
                                    preferred_element_type=jnp.float32)
                    + b2_ref[...][None, :])


def _tc_pool_head(z, st, g, be, batchf, Wfc1, bfc1, Wfc2, bfc2):
  return pl.pallas_call(
      _pool_body,
      grid=(NGRID,),
      in_specs=[pl.BlockSpec((RB, H), lambda i: (i, 0)),
                pl.BlockSpec((2, H), lambda i: (0, 0)),
                pl.BlockSpec((H,), lambda i: (0,)),
                pl.BlockSpec((H,), lambda i: (0,)),
                pl.BlockSpec((1, 1, RB), lambda i: (i, 0, 0)),
                pl.BlockSpec((H, H), lambda i: (0, 0)),
                pl.BlockSpec((H,), lambda i: (0,)),
                pl.BlockSpec((C, H), lambda i: (0, 0)),
                pl.BlockSpec((C,), lambda i: (0,))],
      out_specs=pl.BlockSpec((G, C), lambda i: (0, 0)),
      out_shape=jax.ShapeDtypeStruct((G, C), jnp.float32),
      scratch_shapes=[pltpu.VMEM((G, H), jnp.float32),
                      pltpu.VMEM((G, 128), jnp.float32)],
  )(z, st, g, be, batchf, Wfc1, bfc1, Wfc2, bfc2)


def kernel(x, edge_index, batch, Wl1, Wr1, b1, g1, be1, Wl2, Wr2, b2, g2, be2,
           Wl3, Wr3, b3, g3, be3, Wfc1, bfc1, Wfc2, bfc2):
  src = edge_index[0]
  dst = edge_index[1]
  srcp = jnp.concatenate([src, jnp.zeros((EP - E,), jnp.int32)]
                         ).reshape(EP // CH, CH)
  dstp = jnp.concatenate([dst, jnp.full((EP - E,), N, jnp.int32)]
                         ).reshape(EP // CH, CH)
  zeros2d = jnp.zeros((NP, DC), jnp.float32)
  zeros1d = jnp.zeros((NP,), jnp.float32)

  D0 = x.shape[1]
  xp = jnp.pad(x, ((0, NP - N), (0, 0)))
  xpm = xp.astype(jnp.bfloat16)
  xb_bf = [xpm[:, f * DC:(f + 1) * DC] for f in range(D0 // DC)]

  def _wl_perm(w):
    d = w.shape[1]
    qg = jnp.arange(32).reshape(16, 2).T.reshape(32)
    idx = (32 * jnp.arange(d // 32)[:, None] + qg[None, :]).reshape(d)
    return w[:, idx]
  Wl1, Wl2, Wl3 = _wl_perm(Wl1), _wl_perm(Wl2), _wl_perm(Wl3)
  batchf = (jnp.pad(batch, (0, NP - N), constant_values=G)
            .astype(jnp.float32).reshape(NGRID, 1, RB))

  parts1, deg = _sc_agg(xb_bf, srcp, dstp, zeros2d, zeros1d, with_deg=True)
  z1, st1 = _tc_linear(parts1, deg, xp, Wl1, Wr1, b1)
  h1f, *h1b = _tc_bn_relu(z1, st1, g1, be1)

  parts2 = _sc_agg(h1b, srcp, dstp, zeros2d, zeros1d, with_deg=False)
  z2, st2 = _tc_linear(parts2, deg, h1f, Wl2, Wr2, b2)
  h2f, *h2b = _tc_bn_relu(z2, st2, g2, be2)

  parts3 = _sc_agg(h2b, srcp, dstp, zeros2d, zeros1d, with_deg=False)
  z3, st3 = _tc_linear(parts3, deg, h2f, Wl3, Wr3, b3)
  return _tc_pool_head(z3, st3, g3, be3, batchf, Wfc1, bfc1, Wfc2, bfc2)


# speedup vs baseline: 1.0963x; 1.0116x over previous
import functools

import jax
import jax.numpy as jnp
from jax import lax
from jax.experimental import pallas as pl
from jax.experimental.pallas import tpu as pltpu
from jax.experimental.pallas import tpu_sc as plsc

N = 10000
NP = 10240
E = 160000
EP = 163840
DC = 64
H = 512
C = 128
G = 64
EPS = 1e-5

NTILES = 32
EPT = EP // NTILES
CH = 128
NCH = EPT // CH
RPT = NP // 16
RB = 1280
NGRID = NP // RB


NB = 4
NG = NCH // NB
MASK_HI = -65536


def _sc_agg(tables, srcp2d, dstp2d, zeros2d, zeros1d, with_deg):
  F = len(tables)
  mesh = plsc.VectorSubcoreMesh(core_axis_name="c", subcore_axis_name="s")
  out_type = [jax.ShapeDtypeStruct((2 * NP, DC), jnp.float32) for _ in range(F)]
  if with_deg:
    out_type.append(jax.ShapeDtypeStruct((2 * NP,), jnp.float32))

  scratch_types = (
      [pltpu.VMEM((NCH, CH), jnp.int32),
       pltpu.VMEM((NCH, CH), jnp.int32),
       pltpu.VMEM((CH,), jnp.float32)]
      + [pltpu.VMEM((CH, DC), jnp.bfloat16) for _ in range(NB)]
      + [pltpu.VMEM((CH, DC), jnp.float32) for _ in range(2)]
      + [pltpu.VMEM_SHARED((NP, DC), jnp.float32),
         pltpu.VMEM_SHARED((NP,), jnp.float32)]
      + [pltpu.SemaphoreType.DMA for _ in range(NB)]
      + [pltpu.SemaphoreType.DMA,
         pltpu.SemaphoreType.DMA]
  )

  def body(*refs):
    tabs = refs[:F]
    srcr, dstr, z2, z1 = refs[F:F + 4]
    nin = F + 4
    outs = refs[nin:nin + F]
    degout = refs[nin + F] if with_deg else None
    sc = refs[nin + F + (1 if with_deg else 0):]
    sidx_all, didx_all, ones_v = sc[0:3]
    rows_bf = sc[3:3 + NB]
    payloads = sc[3 + NB:5 + NB]
    agg_sh, deg_sh = sc[5 + NB:7 + NB]
    gsems = sc[7 + NB:7 + 2 * NB]
    ssem, dsem = sc[7 + 2 * NB:9 + 2 * NB]

    cid = lax.axis_index("c")
    sid = lax.axis_index("s")
    wid = sid * 2 + cid
    r0 = sid * RPT

    pltpu.sync_copy(srcr.at[pl.ds(wid * NCH, NCH)], sidx_all)
    pltpu.sync_copy(dstr.at[pl.ds(wid * NCH, NCH)], didx_all)
    if with_deg:
      for k in range(CH // 16):
        ones_v[pl.ds(k * 16, 16)] = jnp.full((16,), 1.0, jnp.float32)

    for f in range(F):
      pltpu.sync_copy(z2.at[pl.ds(r0, RPT)], agg_sh.at[pl.ds(r0, RPT)])
      if with_deg and f == 0:
        pltpu.sync_copy(z1.at[pl.ds(r0, RPT)], deg_sh.at[pl.ds(r0, RPT)])
      plsc.subcore_barrier()

      def group(g, carry):
        gd = []
        for b in range(NB):
          c = g * NB + b
          gd.append(pltpu.async_copy(tabs[f].at[sidx_all.at[c]], rows_bf[b],
                                     gsems[b]))
        psds = [None, None]
        dd = []
        for b in range(NB):
          c = g * NB + b
          pay = payloads[b % 2]
          gd[b].wait()
          if psds[b % 2] is not None:
            psds[b % 2].wait()

          def conv(r8, cc):
            for r0 in range(8):
              for k in range(DC // 32):
                v = rows_bf[b][r8 * 8 + r0, pl.ds(32 * k, 32)]
                vi = plsc.bitcast(v, jnp.int32)
                lo = plsc.bitcast(vi << 16, jnp.float32)
                hi = plsc.bitcast(vi & MASK_HI, jnp.float32)
                pay[r8 * 8 + r0, pl.ds(32 * k, 16)] = lo
                pay[r8 * 8 + r0, pl.ds(32 * k + 16, 16)] = hi
            return cc
          lax.fori_loop(0, CH // 8, conv, 0)

          psds[b % 2] = pltpu.async_copy(pay, agg_sh.at[didx_all.at[c]],
                                         ssem, add=True)
          if with_deg and f == 0:
            dd.append(pltpu.async_copy(ones_v, deg_sh.at[didx_all.at[c]],
                                       dsem, add=True))
        for psd in psds:
          if psd is not None:
            psd.wait()
        for d in dd:
          d.wait()
        return carry

      lax.fori_loop(0, NG, group, 0)
      plsc.subcore_barrier()

      pltpu.sync_copy(agg_sh.at[pl.ds(r0, RPT)],
                      outs[f].at[pl.ds(cid * NP + r0, RPT)])
      if with_deg and f == 0:
        pltpu.sync_copy(deg_sh.at[pl.ds(r0, RPT)],
                        degout.at[pl.ds(cid * NP + r0, RPT)])

  fn = pl.kernel(body, out_type=out_type, mesh=mesh,
                 scratch_types=scratch_types,
                 compiler_params=pltpu.CompilerParams(
                     use_tc_tiling_on_sc=False, needs_layout_passes=False))
  res = fn(*tables, srcp2d, dstp2d, zeros2d, zeros1d)
  if with_deg:
    parts = [r.reshape(2, NP, DC) for r in res[:F]]
    return parts, res[F].reshape(2, NP)
  return [r.reshape(2, NP, DC) for r in res]


def _linear_body(F, *refs):
  parts = refs[:F]
  deg = refs[F]
  h_ref = refs[F + 1]
  wl, wr, b = refs[F + 2:F + 5]
  z_ref, st_ref = refs[F + 5:]

  i = pl.program_id(0)
  d = deg[0, :] + deg[1, :]
  inv = 1.0 / jnp.maximum(d, 1.0)

  dn = (((1,), (1,)), ((), ()))
  pre = jnp.concatenate([parts[f][0] + parts[f][1] for f in range(F)],
                        axis=1) * inv[:, None]
  hcat = h_ref[...]
  acc = (jnp.broadcast_to(b[...][None, :], (RB, H)).astype(jnp.float32)
         + lax.dot_general(pre, wl[...], dn,
                           preferred_element_type=jnp.float32)
         + lax.dot_general(hcat, wr[...], dn,
                           preferred_element_type=jnp.float32))

  rows = i * RB + lax.broadcasted_iota(jnp.int32, (RB, 1), 0)
  z = jnp.where(rows < N, acc, 0.0)
  z_ref[...] = z

  s1 = jnp.sum(z, axis=0)
  s2 = jnp.sum(z * z, axis=0)

  @pl.when(i == 0)
  def _():
    st_ref[...] = jnp.zeros((2, H), jnp.float32)

  st_ref[0, :] += s1
  st_ref[1, :] += s2


def _tc_linear(parts, deg, h, Wl, Wr, b):
  F = len(parts)
  in_specs = (
      [pl.BlockSpec((2, RB, DC), lambda i: (0, i, 0)) for _ in range(F)]
      + [pl.BlockSpec((2, RB), lambda i: (0, i))]
      + [pl.BlockSpec((RB, F * DC), lambda i: (i, 0))]
      + [pl.BlockSpec((H, F * DC), lambda i: (0, 0)),
         pl.BlockSpec((H, F * DC), lambda i: (0, 0)),
         pl.BlockSpec((H,), lambda i: (0,))]
  )
  out_specs = [pl.BlockSpec((RB, H), lambda i: (i, 0)),
               pl.BlockSpec((2, H), lambda i: (0, 0))]
  return pl.pallas_call(
      functools.partial(_linear_body, F),
      grid=(NGRID,),
      in_specs=in_specs,
      out_specs=out_specs,
      out_shape=[jax.ShapeDtypeStruct((NP, H), jnp.float32),
                 jax.ShapeDtypeStruct((2, H), jnp.float32)],
  )(*parts, deg, h, Wl, Wr, b)


def _bn_body(z_ref, st_ref, g_ref, be_ref, h_ref, *out_refs):
  mu = st_ref[0, :] / N
  var = st_ref[1, :] / N - mu * mu
  rstd = lax.rsqrt(var + EPS)
  h = jnp.maximum((z_ref[...] - mu[None, :]) * (rstd * g_ref[...])[None, :]
                  + be_ref[...][None, :], 0.0)
  h_ref[...] = h
  hp = h.astype(jnp.bfloat16)
  for f, o in enumerate(out_refs):
    o[...] = hp[:, f * DC:(f + 1) * DC]


def _tc_bn_relu(z, st, g, be):
  F = H // DC
  return pl.pallas_call(
      _bn_body,
      grid=(NGRID,),
      in_specs=[pl.BlockSpec((RB, H), lambda i: (i, 0)),
                pl.BlockSpec((2, H), lambda i: (0, 0)),
                pl.BlockSpec((H,), lambda i: (0,)),
                pl.BlockSpec((H,), lambda i: (0,))],
      out_specs=([pl.BlockSpec((RB, H), lambda i: (i, 0))]
                 + [pl.BlockSpec((RB, DC), lambda i: (i, 0))
                    for _ in range(F)]),
      out_shape=([jax.ShapeDtypeStruct((NP, H), jnp.float32)]
                 + [jax.ShapeDtypeStruct((NP, DC), jnp.bfloat16)
                    for _ in range(F)]),
  )(z, st, g, be)


def _pool_body(z_ref, st_ref, g_ref, be_ref, b_ref, w1_ref, b1_ref, w2_ref,
               b2_ref, out_ref, pool_ref, cnt_ref):
  i = pl.program_id(0)
  mu = st_ref[0, :] / N
  var = st_ref[1, :] / N - mu * mu
  rstd = lax.rsqrt(var + EPS)
  h = jnp.maximum((z_ref[...] - mu[None, :]) * (rstd * g_ref[...])[None, :]
                  + be_ref[...][None, :], 0.0)

  bvals = b_ref[0, 0, :]
  gids = lax.broadcasted_iota(jnp.int32, (G, RB), 0).astype(jnp.float32)
  oh = jnp.where(gids == bvals[None, :], 1.0, 0.0)

  @pl.when(i == 0)
  def _():
    pool_ref[...] = jnp.zeros((G, H), jnp.float32)
    cnt_ref[...] = jnp.zeros((G, 128), jnp.float32)

  dn = (((1,), (0,)), ((), ()))
  pool_ref[...] += lax.dot_general(oh, h, dn,
                                   preferred_element_type=jnp.float32)
  cnt_ref[...] += jnp.broadcast_to(jnp.sum(oh, axis=1)[:, None], (G, 128))

  @pl.when(i == NGRID - 1)
  def _():
    cnt = cnt_ref[:, 0:1]
    pooled = pool_ref[...] / jnp.maximum(cnt, 1.0)
    dnt = (((1,), (1,)), ((), ()))
    t = jnp.maximum(
        lax.dot_general(pooled, w1_ref[...], dnt,
                        preferred_element_type=jnp.float32)
        + b1_ref[...][None, :], 0.0)
    out_ref[...] = (lax.dot_general(t, w2_ref[...], dnt,
                                    preferred_element_type=jnp.float32)
                    + b2_ref[...][None, :])


def _tc_pool_head(z, st, g, be, batchf, Wfc1, bfc1, Wfc2, bfc2):
  return pl.pallas_call(
      _pool_body,
      grid=(NGRID,),
      in_specs=[pl.BlockSpec((RB, H), lambda i: (i, 0)),
                pl.BlockSpec((2, H), lambda i: (0, 0)),
                pl.BlockSpec((H,), lambda i: (0,)),
                pl.BlockSpec((H,), lambda i: (0,)),
                pl.BlockSpec((1, 1, RB), lambda i: (i, 0, 0)),
                pl.BlockSpec((H, H), lambda i: (0, 0)),
                pl.BlockSpec((H,), lambda i: (0,)),
                pl.BlockSpec((C, H), lambda i: (0, 0)),
                pl.BlockSpec((C,), lambda i: (0,))],
      out_specs=pl.BlockSpec((G, C), lambda i: (0, 0)),
      out_shape=jax.ShapeDtypeStruct((G, C), jnp.float32),
      scratch_shapes=[pltpu.VMEM((G, H), jnp.float32),
                      pltpu.VMEM((G, 128), jnp.float32)],
  )(z, st, g, be, batchf, Wfc1, bfc1, Wfc2, bfc2)


def kernel(x, edge_index, batch, Wl1, Wr1, b1, g1, be1, Wl2, Wr2, b2, g2, be2,
           Wl3, Wr3, b3, g3, be3, Wfc1, bfc1, Wfc2, bfc2):
  src = edge_index[0]
  dst = edge_index[1]
  srcp = jnp.concatenate([src, jnp.zeros((EP - E,), jnp.int32)]
                         ).reshape(EP // CH, CH)
  dstp = jnp.concatenate([dst, jnp.full((EP - E,), N, jnp.int32)]
                         ).reshape(EP // CH, CH)
  zeros2d = jnp.zeros((NP, DC), jnp.float32)
  zeros1d = jnp.zeros((NP,), jnp.float32)

  D0 = x.shape[1]
  xp = jnp.pad(x, ((0, NP - N), (0, 0)))
  xpm = xp.astype(jnp.bfloat16)
  xb_bf = [xpm[:, f * DC:(f + 1) * DC] for f in range(D0 // DC)]

  def _wl_perm(w):
    d = w.shape[1]
    qg = jnp.arange(32).reshape(16, 2).T.reshape(32)
    idx = (32 * jnp.arange(d // 32)[:, None] + qg[None, :]).reshape(d)
    return w[:, idx]
  Wl1, Wl2, Wl3 = _wl_perm(Wl1), _wl_perm(Wl2), _wl_perm(Wl3)
  batchf = (jnp.pad(batch, (0, NP - N), constant_values=G)
            .astype(jnp.float32).reshape(NGRID, 1, RB))

  parts1, deg = _sc_agg(xb_bf, srcp, dstp, zeros2d, zeros1d, with_deg=True)
  z1, st1 = _tc_linear(parts1, deg, xp, Wl1, Wr1, b1)
  h1f, *h1b = _tc_bn_relu(z1, st1, g1, be1)

  parts2 = _sc_agg(h1b, srcp, dstp, zeros2d, zeros1d, with_deg=False)
  z2, st2 = _tc_linear(parts2, deg, h1f, Wl2, Wr2, b2)
  h2f, *h2b = _tc_bn_relu(z2, st2, g2, be2)

  parts3 = _sc_agg(h2b, srcp, dstp, zeros2d, zeros1d, with_deg=False)
  z3, st3 = _tc_linear(parts3, deg, h2f, Wl3, Wr3, b3)
  return _tc_pool_head(z3, st3, g3, be3, batchf, Wfc1, bfc1, Wfc2, bfc2)
